# baseline (speedup 1.0000x reference)
import functools

import jax
import jax.numpy as jnp
from jax import lax
from jax.experimental import pallas as pl
from jax.experimental.pallas import tpu as pltpu
from jax.experimental.pallas import tpu_sc as plsc

_N_SAMPLES = 1024
_N_PIX = 2097152
_H = 4
_N2 = _N_PIX // _H
_NW = 32
_PPW = _N2 // _NW
_CHUNK = 8192
_NCHUNK = _PPW // _CHUNK
_L = 16
_NBLK = _N_PIX // 128
_F4 = _NBLK * 4 * 128


def _crf_table_tc(f0_ref, basis_ref, weight_ref, out_ref):
    out_ref[...] = f0_ref[...] + lax.dot_general(
        weight_ref[...], basis_ref[...],
        (((1,), (0,)), ((), ())), preferred_element_type=jnp.float32)


def _make_sc_half(h):
    def _sc_body(p0_hbm, p1_hbm, p2_hbm, e16_hbm, table_hbm, out_hbm,
                 table_v, e16_v,
                 ib00, ib01, ib02, ib10, ib11, ib12, ob0, ob1,
                 sem_i0, sem_i1, sem_o0, sem_o1, sem_tab):
        planes = (p0_hbm, p1_hbm, p2_hbm)
        wid = lax.axis_index("s") * 2 + lax.axis_index("c")
        p0 = wid * _PPW
        o_half = h * _N2 * 4

        pltpu.async_copy(table_hbm, table_v, sem_tab)
        pltpu.async_copy(e16_hbm, e16_v, sem_tab)

        ibufs = ((ib00, ib01, ib02), (ib10, ib11, ib12))
        obufs = (ob0, ob1)
        sems_i = (sem_i0, sem_i1)
        sems_o = (sem_o0, sem_o1)

        def start_in(g, b):
            pp = p0 + g * _CHUNK
            for c in range(3):
                pltpu.async_copy(planes[c].at[pl.ds(pp, _CHUNK)],
                                 ibufs[b][c], sems_i[b])

        def wait_in(g, b):
            pp = p0 + g * _CHUNK
            for c in range(3):
                pltpu.make_async_copy(planes[c].at[pl.ds(pp, _CHUNK)],
                                      ibufs[b][c], sems_i[b]).wait()

        start_in(0, 0)
        pltpu.make_async_copy(table_hbm, table_v, sem_tab).wait()
        pltpu.make_async_copy(e16_hbm, e16_v, sem_tab).wait()
        scale = e16_v[...] * 1023.0

        def run_inner(ibc, obuf):
            @plsc.parallel_loop(0, _CHUNK // _L, step=1, unroll=2)
            def _(v):
                off = v * _L
                obase = (v // 8) * 512 + (v % 8) * _L
                for c in range(3):
                    x = ibc[c][pl.ds(off, _L)]
                    pos = jnp.minimum(x * scale, 1022.9999389648438)
                    idx = pos.astype(jnp.int32)
                    frac = pos - idx.astype(jnp.float32)
                    t0 = idx + c * _N_SAMPLES
                    y0 = plsc.load_gather(table_v, [t0])
                    y1 = plsc.load_gather(table_v, [t0 + 1])
                    obuf[pl.ds(obase + c * 128, _L)] = y0 + frac * (y1 - y0)

        for g in range(_NCHUNK):
            b = g % 2
            wait_in(g, b)
            if g + 1 < _NCHUNK:
                start_in(g + 1, 1 - b)
            if g >= 2:
                pltpu.make_async_copy(
                    obufs[b],
                    out_hbm.at[pl.ds(o_half + (p0 + (g - 2) * _CHUNK) * 4,
                                     _CHUNK * 4)],
                    sems_o[b]).wait()
            run_inner(ibufs[b], obufs[b])
            pltpu.async_copy(
                obufs[b],
                out_hbm.at[pl.ds(o_half + (p0 + g * _CHUNK) * 4, _CHUNK * 4)],
                sems_o[b])

        for g in (_NCHUNK - 2, _NCHUNK - 1):
            pltpu.make_async_copy(
                obufs[g % 2],
                out_hbm.at[pl.ds(o_half + (p0 + g * _CHUNK) * 4, _CHUNK * 4)],
                sems_o[g % 2]).wait()

    mesh = plsc.VectorSubcoreMesh(core_axis_name="c", subcore_axis_name="s")
    return pl.kernel(
        _sc_body,
        out_type=(),
        mesh=mesh,
        compiler_params=pltpu.CompilerParams(needs_layout_passes=False),
        scratch_types=[
            pltpu.VMEM((3 * _N_SAMPLES,), jnp.float32),
            pltpu.VMEM((_L,), jnp.float32),
            pltpu.VMEM((_CHUNK,), jnp.float32),
            pltpu.VMEM((_CHUNK,), jnp.float32),
            pltpu.VMEM((_CHUNK,), jnp.float32),
            pltpu.VMEM((_CHUNK,), jnp.float32),
            pltpu.VMEM((_CHUNK,), jnp.float32),
            pltpu.VMEM((_CHUNK,), jnp.float32),
            pltpu.VMEM((_CHUNK * 4,), jnp.float32),
            pltpu.VMEM((_CHUNK * 4,), jnp.float32),
            pltpu.SemaphoreType.DMA,
            pltpu.SemaphoreType.DMA,
            pltpu.SemaphoreType.DMA,
            pltpu.SemaphoreType.DMA,
            pltpu.SemaphoreType.DMA,
        ],
    )


@functools.partial(jax.jit, static_argnames=())
def kernel(hdr, exposure, f0, basis, weight):
    crf = pl.pallas_call(
        _crf_table_tc,
        out_shape=jax.ShapeDtypeStruct((3, _N_SAMPLES), jnp.float32),
    )(f0, basis, weight)
    table = crf.reshape(3 * _N_SAMPLES)
    hdr_t = hdr.T
    e16 = jnp.broadcast_to(exposure, (_L,)).astype(jnp.float32)

    out_ref = pl.empty_ref_like(pltpu.HBM((_F4,), jnp.float32))
    srcs = [hdr_t]
    for h in range(1, _H):
        srcs.append(lax.optimization_barrier(srcs[-1]))
    for h in range(_H):
        planes = [srcs[h][c, h * _N2:(h + 1) * _N2] for c in range(3)]
        _make_sc_half(h)(planes[0], planes[1], planes[2], e16, table, out_ref)
    out_flat = out_ref[...]
    out4 = out_flat.reshape(_NBLK, 4, 128).transpose(0, 2, 1)
    return out4.reshape(_N_PIX, 4)[:, :3]


# speedup vs baseline: 1.1099x; 1.1099x over previous
import functools

import jax
import jax.numpy as jnp
from jax import lax
from jax.experimental import pallas as pl
from jax.experimental.pallas import tpu as pltpu
from jax.experimental.pallas import tpu_sc as plsc

_N_SAMPLES = 1024
_N_PIX = 2097152
_H = 2
_N2 = _N_PIX // _H
_NW = 32
_PPW = _N2 // _NW
_CHUNK = 8192
_NCHUNK = _PPW // _CHUNK
_L = 16
_NBLK = _N_PIX // 128
_F4 = _NBLK * 4 * 128


def _crf_table_tc(f0_ref, basis_ref, weight_ref, out_ref):
    out_ref[...] = f0_ref[...] + lax.dot_general(
        weight_ref[...], basis_ref[...],
        (((1,), (0,)), ((), ())), preferred_element_type=jnp.float32)


def _make_sc_half(h):
    def _sc_body(p0_hbm, p1_hbm, p2_hbm, e16_hbm, table_hbm, out_hbm,
                 table_v, e16_v,
                 ib00, ib01, ib02, ib10, ib11, ib12, ob0, ob1,
                 sem_i0, sem_i1, sem_o0, sem_o1, sem_tab):
        planes = (p0_hbm, p1_hbm, p2_hbm)
        wid = lax.axis_index("s") * 2 + lax.axis_index("c")
        p0 = wid * _PPW
        o_half = h * _N2 * 4

        pltpu.async_copy(table_hbm, table_v, sem_tab)
        pltpu.async_copy(e16_hbm, e16_v, sem_tab)

        ibufs = ((ib00, ib01, ib02), (ib10, ib11, ib12))
        obufs = (ob0, ob1)
        sems_i = (sem_i0, sem_i1)
        sems_o = (sem_o0, sem_o1)

        def start_in(g, b):
            pp = p0 + g * _CHUNK
            for c in range(3):
                pltpu.async_copy(planes[c].at[pl.ds(pp, _CHUNK)],
                                 ibufs[b][c], sems_i[b])

        def wait_in(g, b):
            pp = p0 + g * _CHUNK
            for c in range(3):
                pltpu.make_async_copy(planes[c].at[pl.ds(pp, _CHUNK)],
                                      ibufs[b][c], sems_i[b]).wait()

        start_in(0, 0)
        pltpu.make_async_copy(table_hbm, table_v, sem_tab).wait()
        pltpu.make_async_copy(e16_hbm, e16_v, sem_tab).wait()
        scale = e16_v[...] * 1023.0

        def run_inner(ibc, obuf):
            @plsc.parallel_loop(0, _CHUNK // _L, step=1, unroll=2)
            def _(v):
                off = v * _L
                obase = (v // 8) * 512 + (v % 8) * _L
                for c in range(3):
                    x = ibc[c][pl.ds(off, _L)]
                    pos = jnp.minimum(x * scale, 1022.9999389648438)
                    idx = pos.astype(jnp.int32)
                    frac = pos - idx.astype(jnp.float32)
                    t0 = idx + c * _N_SAMPLES
                    y0 = plsc.load_gather(table_v, [t0])
                    y1 = plsc.load_gather(table_v, [t0 + 1])
                    obuf[pl.ds(obase + c * 128, _L)] = y0 + frac * (y1 - y0)

        for g in range(_NCHUNK):
            b = g % 2
            wait_in(g, b)
            if g + 1 < _NCHUNK:
                start_in(g + 1, 1 - b)
            if g >= 2:
                pltpu.make_async_copy(
                    obufs[b],
                    out_hbm.at[pl.ds(o_half + (p0 + (g - 2) * _CHUNK) * 4,
                                     _CHUNK * 4)],
                    sems_o[b]).wait()
            run_inner(ibufs[b], obufs[b])
            pltpu.async_copy(
                obufs[b],
                out_hbm.at[pl.ds(o_half + (p0 + g * _CHUNK) * 4, _CHUNK * 4)],
                sems_o[b])

        for g in (_NCHUNK - 2, _NCHUNK - 1):
            pltpu.make_async_copy(
                obufs[g % 2],
                out_hbm.at[pl.ds(o_half + (p0 + g * _CHUNK) * 4, _CHUNK * 4)],
                sems_o[g % 2]).wait()

    mesh = plsc.VectorSubcoreMesh(core_axis_name="c", subcore_axis_name="s")
    return pl.kernel(
        _sc_body,
        out_type=(),
        mesh=mesh,
        compiler_params=pltpu.CompilerParams(needs_layout_passes=False),
        scratch_types=[
            pltpu.VMEM((3 * _N_SAMPLES,), jnp.float32),
            pltpu.VMEM((_L,), jnp.float32),
            pltpu.VMEM((_CHUNK,), jnp.float32),
            pltpu.VMEM((_CHUNK,), jnp.float32),
            pltpu.VMEM((_CHUNK,), jnp.float32),
            pltpu.VMEM((_CHUNK,), jnp.float32),
            pltpu.VMEM((_CHUNK,), jnp.float32),
            pltpu.VMEM((_CHUNK,), jnp.float32),
            pltpu.VMEM((_CHUNK * 4,), jnp.float32),
            pltpu.VMEM((_CHUNK * 4,), jnp.float32),
            pltpu.SemaphoreType.DMA,
            pltpu.SemaphoreType.DMA,
            pltpu.SemaphoreType.DMA,
            pltpu.SemaphoreType.DMA,
            pltpu.SemaphoreType.DMA,
        ],
    )


@functools.partial(jax.jit, static_argnames=())
def kernel(hdr, exposure, f0, basis, weight):
    crf = pl.pallas_call(
        _crf_table_tc,
        out_shape=jax.ShapeDtypeStruct((3, _N_SAMPLES), jnp.float32),
    )(f0, basis, weight)
    table = crf.reshape(3 * _N_SAMPLES)
    hdr_t = hdr.T
    e16 = jnp.broadcast_to(exposure, (_L,)).astype(jnp.float32)

    out_ref = pl.empty_ref_like(pltpu.HBM((_F4,), jnp.float32))
    srcs = [hdr_t]
    for h in range(1, _H):
        srcs.append(lax.optimization_barrier(srcs[-1]))
    for h in range(_H):
        planes = [srcs[h][c, h * _N2:(h + 1) * _N2] for c in range(3)]
        _make_sc_half(h)(planes[0], planes[1], planes[2], e16, table, out_ref)
    out_flat = out_ref[...]
    out4 = out_flat.reshape(_NBLK, 4, 128).transpose(0, 2, 1)
    return out4.reshape(_N_PIX, 4)[:, :3]
